# Initial kernel scaffold; baseline (speedup 1.0000x reference)
import jax
import jax.numpy as jnp
from jax.experimental import pallas as pl


def kernel(x_comment, x_topic, x_claim, edge_src_sim, edge_dst_sim, edge_src_about, edge_dst_about, edge_src_targets, edge_dst_targets, W1_sim_l, b1_sim, W1_sim_r, W1_ab_l, b1_ab, W1_ab_r, W1_tg_l, b1_tg, W1_tg_r, W2_sim_l, b2_sim, W2_sim_r, W2_ab_l, b2_ab, W2_ab_r, W2_tg_l, b2_tg, W2_tg_r, Wt, bt, Wi, bi, Wm, bm):
    raise NotImplementedError("write your pallas kernel here")


# speedup vs baseline: 1.0240x; 1.0240x over previous
import jax
import jax.numpy as jnp
from jax import lax
from jax.experimental import pallas as pl

H = 128
N_C = 10000
NPAD = 10240

RBLK = 512
GRID = NPAD // RBLK


def _mean(a, c):
    return a[...] / jnp.maximum(c[...], 1.0)


def _dot(a, b):
    return jnp.dot(a, b, preferred_element_type=jnp.float32)


def _tc1_body(as_, cs, aa, ca, at, ct,
              x, Wsl, Wal, Wtl, Wr, b, out):
    acc = (_dot(_mean(as_, cs), Wsl[...])
           + _dot(_mean(aa, ca), Wal[...])
           + _dot(_mean(at, ct), Wtl[...])
           + _dot(x[...], Wr[...]) + b[...])
    out[...] = jnp.maximum(acc, 0.0)


def _tc2_body(a2, cs, aa, ca, at, ct,
              c1, Wsl, Wal, Wtl, Wr, b,
              Whead, bhead, Wmh, Wmti, bmp, out):
    h = jnp.maximum(_dot(_mean(a2, cs), Wsl[...])
                    + _dot(_mean(aa, ca), Wal[...])
                    + _dot(_mean(at, ct), Wtl[...])
                    + _dot(c1[...], Wr[...]) + b[...], 0.0)
    logits = _dot(h, Whead[...]) + bhead[...]
    lane = lax.broadcasted_iota(jnp.int32, (RBLK, H), 1)
    mt = lane < 6
    mi = (lane >= 6) & (lane < 10)
    tp = jnp.where(mt, jax.nn.sigmoid(logits), 0.0)
    z = jnp.where(mi, logits, -1e30)
    e = jnp.where(mi, jnp.exp(z - jnp.max(z, axis=-1, keepdims=True)), 0.0)
    ip = e / jnp.sum(e, axis=-1, keepdims=True)
    impl = _dot(h, Wmh[...]) + _dot(tp + ip, Wmti[...]) + bmp[...]
    out[...] = jnp.where(lane < 10, logits, 0.0) + impl


def _row_spec():
    return pl.BlockSpec((RBLK, H), lambda i: (i, 0))


def _cnt_spec():
    return pl.BlockSpec((RBLK, 1), lambda i: (i, 0))


def _w_spec():
    return pl.BlockSpec((H, H), lambda i: (0, 0))


def _b_spec():
    return pl.BlockSpec((1, H), lambda i: (0, 0))


_tc1 = pl.pallas_call(
    _tc1_body,
    grid=(GRID,),
    in_specs=[_row_spec(), _cnt_spec()] * 3 + [_row_spec()]
    + [_w_spec()] * 4 + [_b_spec()],
    out_specs=_row_spec(),
    out_shape=jax.ShapeDtypeStruct((NPAD, H), jnp.float32),
)

_tc2 = pl.pallas_call(
    _tc2_body,
    grid=(GRID,),
    in_specs=[_row_spec(), _cnt_spec()] * 3 + [_row_spec()]
    + [_w_spec()] * 4 + [_b_spec()]
    + [_w_spec(), _b_spec(), _w_spec(), _w_spec(), _b_spec()],
    out_specs=_row_spec(),
    out_shape=jax.ShapeDtypeStruct((NPAD, H), jnp.float32),
)


def _agg(x_src, src, dst):
    return jnp.zeros((NPAD, H), jnp.float32).at[dst].add(x_src[src])


def _cnt(dst, n_edges):
    return (jnp.zeros((NPAD, 1), jnp.float32)
            .at[dst, 0].add(jnp.ones((n_edges,), jnp.float32)))


@jax.jit
def kernel(x_comment, x_topic, x_claim,
           edge_src_sim, edge_dst_sim, edge_src_about, edge_dst_about,
           edge_src_targets, edge_dst_targets,
           W1_sim_l, b1_sim, W1_sim_r, W1_ab_l, b1_ab, W1_ab_r,
           W1_tg_l, b1_tg, W1_tg_r,
           W2_sim_l, b2_sim, W2_sim_r, W2_ab_l, b2_ab, W2_ab_r,
           W2_tg_l, b2_tg, W2_tg_r,
           Wt, bt, Wi, bi, Wm, bm):
    f32 = jnp.float32
    ss, ds = edge_src_sim.astype(jnp.int32), edge_dst_sim.astype(jnp.int32)
    sa, da = edge_src_about.astype(jnp.int32), edge_dst_about.astype(jnp.int32)
    st, dt = (edge_src_targets.astype(jnp.int32),
              edge_dst_targets.astype(jnp.int32))

    agg_sim = _agg(x_comment, ss, ds)
    agg_ab = _agg(x_topic, sa, da)
    agg_tg = _agg(x_claim, st, dt)
    cnt_sim = _cnt(ds, ds.shape[0])
    cnt_ab = _cnt(da, da.shape[0])
    cnt_tg = _cnt(dt, dt.shape[0])

    xp = jnp.zeros((NPAD, H), f32).at[:N_C].set(x_comment)
    Wr1 = W1_sim_r + W1_ab_r + W1_tg_r
    b1 = (b1_sim + b1_ab + b1_tg).reshape(1, H)
    c1 = _tc1(agg_sim, cnt_sim, agg_ab, cnt_ab, agg_tg, cnt_tg, xp,
              W1_sim_l, W1_ab_l, W1_tg_l, Wr1, b1)

    agg_sim2 = _agg(c1, ss, ds)

    Wr2 = W2_sim_r + W2_ab_r + W2_tg_r
    b2 = (b2_sim + b2_ab + b2_tg).reshape(1, H)
    Whead = jnp.zeros((H, H), f32).at[:, 0:6].set(Wt).at[:, 6:10].set(Wi)
    bhead = jnp.zeros((1, H), f32).at[0, 0:6].set(bt).at[0, 6:10].set(bi)
    Wmh = jnp.zeros((H, H), f32).at[:, 10:13].set(Wm[:H])
    Wmti = (jnp.zeros((H, H), f32).at[0:6, 10:13].set(Wm[H:H + 6])
            .at[6:10, 10:13].set(Wm[H + 6:H + 10]))
    bmp = jnp.zeros((1, H), f32).at[0, 10:13].set(bm)

    out = _tc2(agg_sim2, cnt_sim, agg_ab, cnt_ab, agg_tg, cnt_tg, c1,
               W2_sim_l, W2_ab_l, W2_tg_l, Wr2, b2,
               Whead, bhead, Wmh, Wmti, bmp)

    return (out[:N_C, 0:6], out[:N_C, 6:10], out[:N_C, 10:13])
